# baseline (speedup 1.0000x reference)
import functools

import jax
import jax.numpy as jnp
from jax.experimental import pallas as pl
from jax.experimental.pallas import tpu as pltpu

B, S, D, DS = 4, 2048, 1024, 64
NF, NR, NT = 64, 32, 48
TKF, TKR, TKT = 8, 4, 6

BLK = 512
NS = S // BLK


def _topk_mask_normalize(w, k, n):
    iota = jax.lax.broadcasted_iota(jnp.int32, (1, n), 1)
    wm = w
    sel = jnp.zeros((1, n), dtype=jnp.bool_)
    for _ in range(k):
        mx = jnp.max(wm, axis=1, keepdims=True)
        eq = wm == mx
        idx = jnp.min(jnp.where(eq, iota, n), axis=1, keepdims=True)
        hit = iota == idx
        sel = jnp.logical_or(sel, hit)
        wm = jnp.where(hit, -jnp.inf, wm)
    sparse = jnp.where(sel, w, 0.0)
    return sparse / (jnp.sum(sparse, axis=1, keepdims=True) + 1e-8)


def _router_kernel(x_ref, imp_ref, w_ref, b_ref,
                   embf_ref, embr_ref, embt_ref,
                   of_ref, or_ref, ot_ref,
                   accf, accr, acct):
    s = pl.program_id(1)

    xb = x_ref[0]
    h = jnp.dot(xb, w_ref[...], preferred_element_type=jnp.float32)
    h = h + b_ref[...]
    imp = imp_ref[0]

    @pl.when(s == 0)
    def _():
        accf[...] = jnp.zeros_like(accf)
        accr[...] = jnp.zeros_like(accr)
        acct[...] = jnp.zeros_like(acct)

    def pool(emb_t_ref, acc):
        et = emb_t_ref[...]
        nrm = jnp.sqrt(jnp.sum(et * et, axis=0, keepdims=True))
        et = et / (nrm + 1e-12)
        logits = jnp.dot(h, et, preferred_element_type=jnp.float32)
        m = jnp.max(logits, axis=1, keepdims=True)
        e = jnp.exp(logits - m)
        p = e / jnp.sum(e, axis=1, keepdims=True)
        acc[...] += jnp.sum(p * imp, axis=0, keepdims=True)

    pool(embf_ref, accf)
    pool(embr_ref, accr)
    pool(embt_ref, acct)

    @pl.when(s == NS - 1)
    def _():
        of_ref[0] = _topk_mask_normalize(accf[...], TKF, NF)
        or_ref[0] = _topk_mask_normalize(accr[...], TKR, NR)
        ot_ref[0] = _topk_mask_normalize(acct[...], TKT, NT)


@functools.partial(jax.jit, static_argnames=("interpret",))
def kernel(x, importance, W_proj, b_proj, neuron_emb, interpret=False):
    imp3 = importance.reshape(B, S, 1)
    b2 = b_proj.reshape(1, DS)
    emb_t = neuron_emb.T
    embf_t = emb_t[:, :NF]
    embr_t = emb_t[:, NF:NF + NR]
    embt_t = emb_t[:, NF + NR:]

    of, orr, ot = pl.pallas_call(
        _router_kernel,
        grid=(B, NS),
        in_specs=[
            pl.BlockSpec((1, BLK, D), lambda b, s: (b, s, 0)),
            pl.BlockSpec((1, BLK, 1), lambda b, s: (b, s, 0)),
            pl.BlockSpec((D, DS), lambda b, s: (0, 0)),
            pl.BlockSpec((1, DS), lambda b, s: (0, 0)),
            pl.BlockSpec((DS, NF), lambda b, s: (0, 0)),
            pl.BlockSpec((DS, NR), lambda b, s: (0, 0)),
            pl.BlockSpec((DS, NT), lambda b, s: (0, 0)),
        ],
        out_specs=[
            pl.BlockSpec((1, 1, NF), lambda b, s: (b, 0, 0)),
            pl.BlockSpec((1, 1, NR), lambda b, s: (b, 0, 0)),
            pl.BlockSpec((1, 1, NT), lambda b, s: (b, 0, 0)),
        ],
        out_shape=[
            jax.ShapeDtypeStruct((B, 1, NF), jnp.float32),
            jax.ShapeDtypeStruct((B, 1, NR), jnp.float32),
            jax.ShapeDtypeStruct((B, 1, NT), jnp.float32),
        ],
        scratch_shapes=[
            pltpu.VMEM((1, NF), jnp.float32),
            pltpu.VMEM((1, NR), jnp.float32),
            pltpu.VMEM((1, NT), jnp.float32),
        ],
        compiler_params=pltpu.CompilerParams(
            dimension_semantics=("parallel", "arbitrary"),
        ),
        interpret=interpret,
    )(x, imp3, W_proj, b2, embf_t, embr_t, embt_t)

    of, orr, ot = of[:, 0], orr[:, 0], ot[:, 0]
    return jnp.concatenate([of, orr, orr, ot], axis=-1)


# speedup vs baseline: 1.6029x; 1.6029x over previous
import functools

import jax
import jax.numpy as jnp
import numpy as np
from jax.experimental import pallas as pl
from jax.experimental.pallas import tpu as pltpu

B, S, D, DS = 4, 2048, 1024, 64
NF, NR, NT = 64, 32, 48
N_ALL = NF + NR + NT
TKF, TKR, TKT = 8, 4, 6

BLK = 1024
NS = S // BLK
NSEG = 8


def _topk_mask_normalize(w, k, n):
    wt = jnp.swapaxes(w, 0, 1)
    il = jax.lax.broadcasted_iota(jnp.int32, (1, n), 1)
    jt = jax.lax.broadcasted_iota(jnp.int32, (n, 1), 0)
    beats = (wt > w) | ((wt == w) & (jt < il))
    rank = jnp.sum(beats.astype(jnp.float32), axis=0, keepdims=True)
    sparse = jnp.where(rank < k, w, 0.0)
    return sparse / (jnp.sum(sparse, axis=1, keepdims=True) + 1e-8)


def _router_kernel(x_ref, imp_ref, w_ref, b_ref, embt_ref,
                   msum_ref, mbc_ref,
                   of_ref, or_ref, ot_ref, acc):
    s = pl.program_id(1)

    xb = x_ref[0].astype(jnp.bfloat16)
    h = jnp.dot(xb, w_ref[...], preferred_element_type=jnp.float32)
    h = h + b_ref[...]
    imp = imp_ref[0]

    et = embt_ref[...]
    nrm = jnp.sqrt(jnp.sum(et * et, axis=0, keepdims=True))
    et_bf = (et / (nrm + 1e-12)).astype(jnp.bfloat16)

    logits = jnp.dot(h.astype(jnp.bfloat16), et_bf,
                     preferred_element_type=jnp.float32)
    e = jnp.exp(logits)
    sseg = jnp.dot(e, msum_ref[...],
                   preferred_element_type=jnp.float32)
    winv = imp / sseg
    wbc = jnp.dot(winv, mbc_ref[...],
                  preferred_element_type=jnp.float32)
    part = jnp.sum(e * wbc, axis=0, keepdims=True)

    @pl.when(s == 0)
    def _():
        acc[...] = jnp.zeros_like(acc)

    acc[...] += part

    @pl.when(s == NS - 1)
    def _():
        a = acc[...]
        of_ref[0] = _topk_mask_normalize(a[:, :NF], TKF, NF)
        or_ref[0] = _topk_mask_normalize(a[:, NF:NF + NR], TKR, NR)
        ot_ref[0] = _topk_mask_normalize(a[:, NF + NR:], TKT, NT)


@functools.partial(jax.jit, static_argnames=("interpret",))
def kernel(x, importance, W_proj, b_proj, neuron_emb, interpret=False):
    imp3 = importance.reshape(B, S, 1)
    b2 = b_proj.reshape(1, DS)
    emb_t = neuron_emb.T
    w_bf = W_proj.astype(jnp.bfloat16)

    seg_id = np.zeros((N_ALL,), dtype=np.int64)
    seg_id[NF:NF + NR] = 1
    seg_id[NF + NR:] = 2
    msum = np.zeros((N_ALL, NSEG), dtype=np.float32)
    msum[np.arange(N_ALL), seg_id] = 1.0
    msum[0, 3:] = 1.0
    mbc = np.zeros((NSEG, N_ALL), dtype=np.float32)
    mbc[seg_id, np.arange(N_ALL)] = 1.0

    of, orr, ot = pl.pallas_call(
        _router_kernel,
        grid=(B, NS),
        in_specs=[
            pl.BlockSpec((1, BLK, D), lambda b, s: (b, s, 0)),
            pl.BlockSpec((1, BLK, 1), lambda b, s: (b, s, 0)),
            pl.BlockSpec((D, DS), lambda b, s: (0, 0)),
            pl.BlockSpec((1, DS), lambda b, s: (0, 0)),
            pl.BlockSpec((DS, N_ALL), lambda b, s: (0, 0)),
            pl.BlockSpec((N_ALL, NSEG), lambda b, s: (0, 0)),
            pl.BlockSpec((NSEG, N_ALL), lambda b, s: (0, 0)),
        ],
        out_specs=[
            pl.BlockSpec((1, 1, NF), lambda b, s: (b, 0, 0)),
            pl.BlockSpec((1, 1, NR), lambda b, s: (b, 0, 0)),
            pl.BlockSpec((1, 1, NT), lambda b, s: (b, 0, 0)),
        ],
        out_shape=[
            jax.ShapeDtypeStruct((B, 1, NF), jnp.float32),
            jax.ShapeDtypeStruct((B, 1, NR), jnp.float32),
            jax.ShapeDtypeStruct((B, 1, NT), jnp.float32),
        ],
        scratch_shapes=[
            pltpu.VMEM((1, N_ALL), jnp.float32),
        ],
        compiler_params=pltpu.CompilerParams(
            dimension_semantics=("parallel", "arbitrary"),
        ),
        interpret=interpret,
    )(x, imp3, w_bf, b2, emb_t, jnp.asarray(msum), jnp.asarray(mbc))

    of, orr, ot = of[:, 0], orr[:, 0], ot[:, 0]
    return jnp.concatenate([of, orr, orr, ot], axis=-1)


# speedup vs baseline: 1.7018x; 1.0617x over previous
import functools

import jax
import jax.numpy as jnp
import numpy as np
from jax.experimental import pallas as pl
from jax.experimental.pallas import tpu as pltpu

B, S, D, DS = 4, 2048, 1024, 64
NF, NR, NT = 64, 32, 48
N_ALL = NF + NR + NT
TKF, TKR, TKT = 8, 4, 6

BLK = 1024
NS = S // BLK
NSEG = 8


def _topk_mask_normalize(wc, k, n, base):
    wt = jnp.swapaxes(wc, 0, 1)
    il = jax.lax.broadcasted_iota(jnp.int32, (1, n), 1)
    jt = jax.lax.broadcasted_iota(jnp.int32, (n, 1), 0)
    beats = (wt > wc) | ((wt == wc) & (jt < il))
    rank = jnp.sum(beats.astype(jnp.float32), axis=0, keepdims=True)
    sparse = jnp.where(rank < k, wc + base, 0.0)
    return sparse / (jnp.sum(sparse, axis=1, keepdims=True) + 1e-8)


def _router_kernel(x_ref, imp_ref, w_ref, b_ref, embt_ref,
                   msum_ref, mbc_ref, cbc_ref,
                   of_ref, or_ref, ot_ref, acc, accimp, etn):
    s = pl.program_id(1)

    @pl.when(s == 0)
    def _():
        et = embt_ref[...]
        nrm = jnp.sqrt(jnp.sum(et * et, axis=0, keepdims=True))
        etn[...] = et / (nrm + 1e-12)
        acc[...] = jnp.zeros_like(acc)
        accimp[...] = jnp.zeros_like(accimp)

    xb = x_ref[0]
    h = jnp.dot(xb, w_ref[...], preferred_element_type=jnp.float32)
    h = h + b_ref[...]
    imp = imp_ref[0]

    logits = jnp.dot(h, etn[...],
                     preferred_element_type=jnp.float32)
    e = jnp.exp(logits)
    sseg = jnp.dot(e, msum_ref[...],
                   preferred_element_type=jnp.float32)
    winv = 1.0 / sseg
    wbc = jnp.dot(winv, mbc_ref[...],
                  preferred_element_type=jnp.float32)
    tc = e * wbc - cbc_ref[...]
    part = jnp.dot(imp, tc, preferred_element_type=jnp.float32)

    acc[...] += part
    accimp[...] += jnp.sum(imp, axis=1, keepdims=True)

    @pl.when(s == NS - 1)
    def _():
        a = acc[...]
        si = accimp[...]
        of_ref[0] = _topk_mask_normalize(a[:, :NF], TKF, NF, si / NF)
        or_ref[0] = _topk_mask_normalize(a[:, NF:NF + NR], TKR, NR, si / NR)
        ot_ref[0] = _topk_mask_normalize(a[:, NF + NR:], TKT, NT, si / NT)


@functools.partial(jax.jit, static_argnames=("interpret",))
def kernel(x, importance, W_proj, b_proj, neuron_emb, interpret=False):
    imp3 = importance.reshape(B, 1, S)
    b2 = b_proj.reshape(1, DS)
    emb_t = neuron_emb.T

    seg_id = np.zeros((N_ALL,), dtype=np.int64)
    seg_id[NF:NF + NR] = 1
    seg_id[NF + NR:] = 2
    msum = np.zeros((N_ALL, NSEG), dtype=np.float32)
    msum[np.arange(N_ALL), seg_id] = 1.0
    msum[0, 3:] = 1.0
    mbc = np.zeros((NSEG, N_ALL), dtype=np.float32)
    mbc[seg_id, np.arange(N_ALL)] = 1.0
    seg_n = np.array([NF, NR, NT], dtype=np.float32)
    cbc = (1.0 / seg_n[seg_id]).reshape(1, N_ALL)

    of, orr, ot = pl.pallas_call(
        _router_kernel,
        grid=(B, NS),
        in_specs=[
            pl.BlockSpec((1, BLK, D), lambda b, s: (b, s, 0)),
            pl.BlockSpec((1, 1, BLK), lambda b, s: (b, 0, s)),
            pl.BlockSpec((D, DS), lambda b, s: (0, 0)),
            pl.BlockSpec((1, DS), lambda b, s: (0, 0)),
            pl.BlockSpec((DS, N_ALL), lambda b, s: (0, 0)),
            pl.BlockSpec((N_ALL, NSEG), lambda b, s: (0, 0)),
            pl.BlockSpec((NSEG, N_ALL), lambda b, s: (0, 0)),
            pl.BlockSpec((1, N_ALL), lambda b, s: (0, 0)),
        ],
        out_specs=[
            pl.BlockSpec((1, 1, NF), lambda b, s: (b, 0, 0)),
            pl.BlockSpec((1, 1, NR), lambda b, s: (b, 0, 0)),
            pl.BlockSpec((1, 1, NT), lambda b, s: (b, 0, 0)),
        ],
        out_shape=[
            jax.ShapeDtypeStruct((B, 1, NF), jnp.float32),
            jax.ShapeDtypeStruct((B, 1, NR), jnp.float32),
            jax.ShapeDtypeStruct((B, 1, NT), jnp.float32),
        ],
        scratch_shapes=[
            pltpu.VMEM((1, N_ALL), jnp.float32),
            pltpu.VMEM((1, 1), jnp.float32),
            pltpu.VMEM((DS, N_ALL), jnp.float32),
        ],
        compiler_params=pltpu.CompilerParams(
            dimension_semantics=("parallel", "arbitrary"),
        ),
        interpret=interpret,
    )(x, imp3, W_proj, b2, emb_t,
      jnp.asarray(msum), jnp.asarray(mbc), jnp.asarray(cbc))

    of, orr, ot = of[:, 0], orr[:, 0], ot[:, 0]
    return jnp.concatenate([of, orr, orr, ot], axis=-1)
